# baseline (speedup 1.0000x reference)
import functools

import jax
import jax.numpy as jnp
from jax import lax
from jax.experimental import pallas as pl
from jax.experimental.pallas import tpu as pltpu
from jax.experimental.pallas import tpu_sc as plsc

_NF = 26
_V = 100000
_D = 64
_B = 16384
_NC, _NS = 2, 16
_NW = _NC * _NS
_BPW = _B // _NW
_C = 256

_BS = 400
_NBLK = _V // _BS

_mesh = plsc.VectorSubcoreMesh(core_axis_name="c", subcore_axis_name="s")


def _compact_body(*refs):
    ins = refs[:_NF]
    out = refs[_NF]
    for f in range(_NF):
        out[f, :, 0:_D] = ins[f][...]


_compact_tc = pl.pallas_call(
    _compact_body,
    grid=(_NBLK,),
    in_specs=[pl.BlockSpec((_BS, _D), lambda j: (j, 0))] * _NF,
    out_specs=pl.BlockSpec((_NF, _BS, 2 * _D), lambda j: (0, j, 0)),
    out_shape=jax.ShapeDtypeStruct((_NF, _V, 2 * _D), jnp.float32),
)


@functools.partial(
    pl.kernel,
    out_type=jax.ShapeDtypeStruct((_B, _NF * _D), jnp.float32),
    mesh=_mesh,
    compiler_params=pltpu.CompilerParams(use_tc_tiling_on_sc=False),
    scratch_types=[
        pltpu.VMEM((_NF, _BPW), jnp.int32),
        pltpu.VMEM((2, _C, 2 * _D), jnp.float32),
        pltpu.SemaphoreType.DMA,
        pltpu.SemaphoreType.DMA,
    ],
)
def _gather_kernel(xTs, rows, out, idx_v, rows_v, sem0, sem1):
    sems = (sem0, sem1)

    wid = lax.axis_index("s") * _NC + lax.axis_index("c")
    base = wid * _BPW

    pltpu.sync_copy(xTs.at[:, pl.ds(base, _BPW)], idx_v)

    tasks = [(f, c) for f in range(_NF) for c in range(_BPW // _C)]

    def start(t, b):
        f, c = tasks[t]
        return pltpu.async_copy(
            rows.at[idx_v.at[f, pl.ds(c * _C, _C)]], rows_v.at[b], sems[b])

    copies = [None, None]
    copies[0] = start(0, 0)
    for t in range(len(tasks)):
        b = t % 2
        if t + 1 < len(tasks):
            copies[(t + 1) % 2] = start(t + 1, (t + 1) % 2)
        copies[b].wait()
        f, c = tasks[t]
        pltpu.sync_copy(
            rows_v.at[b, :, pl.ds(0, _D)],
            out.at[pl.ds(base + c * _C, _C), pl.ds(f * _D, _D)])


def kernel(x, table_0, table_1, table_2, table_3, table_4, table_5,
           table_6, table_7, table_8, table_9, table_10, table_11,
           table_12, table_13, table_14, table_15, table_16, table_17,
           table_18, table_19, table_20, table_21, table_22, table_23,
           table_24, table_25):
    tables = (table_0, table_1, table_2, table_3, table_4, table_5,
              table_6, table_7, table_8, table_9, table_10, table_11,
              table_12, table_13, table_14, table_15, table_16, table_17,
              table_18, table_19, table_20, table_21, table_22, table_23,
              table_24, table_25)
    slab = _compact_tc(*tables)
    rows = slab.reshape(_NF * _V, 2 * _D)
    offs = jnp.arange(_NF, dtype=jnp.int32) * _V
    xTs = x.T + offs[:, None]
    return _gather_kernel(xTs, rows)


# speedup vs baseline: 1.0598x; 1.0598x over previous
import functools

import jax
import jax.numpy as jnp
from jax import lax
from jax.experimental import pallas as pl
from jax.experimental.pallas import tpu as pltpu
from jax.experimental.pallas import tpu_sc as plsc

_NF = 26
_V = 100000
_D = 64
_B = 16384
_NC, _NS = 2, 16
_NW = _NC * _NS
_BPW = _B // _NW
_C = 128

_BS = 400
_NBLK = _V // _BS

_mesh = plsc.VectorSubcoreMesh(core_axis_name="c", subcore_axis_name="s")


def _compact_body(*refs):
    ins = refs[:_NF]
    out = refs[_NF]
    for f in range(_NF):
        out[f, :, 0:_D] = ins[f][...]


_compact_tc = pl.pallas_call(
    _compact_body,
    grid=(_NBLK,),
    in_specs=[pl.BlockSpec((_BS, _D), lambda j: (j, 0))] * _NF,
    out_specs=pl.BlockSpec((_NF, _BS, 2 * _D), lambda j: (0, j, 0)),
    out_shape=jax.ShapeDtypeStruct((_NF, _V, 2 * _D), jnp.float32),
)


@functools.partial(
    pl.kernel,
    out_type=jax.ShapeDtypeStruct((_B, _NF * _D), jnp.float32),
    mesh=_mesh,
    scratch_types=[
        pltpu.VMEM((_NF, _BPW), jnp.int32),
        pltpu.VMEM((2, 2, _C, 2 * _D), jnp.float32),
        pltpu.VMEM_SHARED((_NS, _C, 2 * _D), jnp.float32),
        pltpu.SemaphoreType.DMA,
        pltpu.SemaphoreType.DMA,
        pltpu.SemaphoreType.DMA,
    ],
)
def _gather_kernel(xT, slab, out, idx_v, dst, shared, sem0, sem1, semm):
    sems = (sem0, sem1)

    sid = lax.axis_index("s")
    wid = sid * _NC + lax.axis_index("c")
    base = wid * _BPW
    sp = shared.at[sid]

    pltpu.sync_copy(xT.at[:, pl.ds(base, _BPW)], idx_v)

    tasks = [(p, c) for p in range(_NF // 2) for c in range(_BPW // _C)]

    def start(t, b):
        p, c = tasks[t]
        cp0 = pltpu.async_copy(
            slab.at[2 * p].at[idx_v.at[2 * p, pl.ds(c * _C, _C)]],
            dst.at[b, 0], sems[b])
        cp1 = pltpu.async_copy(
            slab.at[2 * p + 1].at[idx_v.at[2 * p + 1, pl.ds(c * _C, _C)]],
            dst.at[b, 1], sems[b])
        return (cp0, cp1)

    copies = [None, None]
    copies[0] = start(0, 0)
    for t in range(len(tasks)):
        b = t % 2
        if t + 1 < len(tasks):
            copies[(t + 1) % 2] = start(t + 1, (t + 1) % 2)
        copies[b][0].wait()
        copies[b][1].wait()
        m0 = pltpu.async_copy(dst.at[b, 0, :, pl.ds(0, _D)],
                              sp.at[:, pl.ds(0, _D)], semm)
        m1 = pltpu.async_copy(dst.at[b, 1, :, pl.ds(0, _D)],
                              sp.at[:, pl.ds(_D, _D)], semm)
        m0.wait()
        m1.wait()
        p, c = tasks[t]
        pltpu.sync_copy(
            sp, out.at[pl.ds(base + c * _C, _C), pl.ds(p * 2 * _D, 2 * _D)])


def kernel(x, table_0, table_1, table_2, table_3, table_4, table_5,
           table_6, table_7, table_8, table_9, table_10, table_11,
           table_12, table_13, table_14, table_15, table_16, table_17,
           table_18, table_19, table_20, table_21, table_22, table_23,
           table_24, table_25):
    tables = (table_0, table_1, table_2, table_3, table_4, table_5,
              table_6, table_7, table_8, table_9, table_10, table_11,
              table_12, table_13, table_14, table_15, table_16, table_17,
              table_18, table_19, table_20, table_21, table_22, table_23,
              table_24, table_25)
    slab = _compact_tc(*tables)
    return _gather_kernel(x.T, slab)


# speedup vs baseline: 2.5472x; 2.4035x over previous
import functools

import jax
import jax.numpy as jnp
from jax import lax
from jax.experimental import pallas as pl
from jax.experimental.pallas import tpu as pltpu
from jax.experimental.pallas import tpu_sc as plsc

_NF = 26
_V = 100000
_D = 64
_B = 16384
_NC, _NS = 2, 16
_NW = _NC * _NS
_BPW = _B // _NW
_C = 128

_BS = 512
_NBLK = (_V + _BS - 1) // _BS

_mesh = plsc.VectorSubcoreMesh(core_axis_name="c", subcore_axis_name="s")


def _compact_body(*refs):
    ins = refs[:_NF]
    out = refs[_NF]
    for f in range(_NF):
        out[f, :, 0:_D] = ins[f][...].T


_compact_tc = pl.pallas_call(
    _compact_body,
    grid=(_NBLK,),
    in_specs=[pl.BlockSpec((_D, _BS), lambda j: (0, j))] * _NF,
    out_specs=pl.BlockSpec((_NF, _BS, 2 * _D), lambda j: (0, j, 0)),
    out_shape=jax.ShapeDtypeStruct((_NF, _V, 2 * _D), jnp.float32),
)


@functools.partial(
    pl.kernel,
    out_type=jax.ShapeDtypeStruct((_B, _NF * _D), jnp.float32),
    mesh=_mesh,
    scratch_types=[
        pltpu.VMEM((_NF, _BPW), jnp.int32),
        pltpu.VMEM((2, 2, _C, 2 * _D), jnp.float32),
        pltpu.VMEM_SHARED((_NS, _C, 2 * _D), jnp.float32),
        pltpu.SemaphoreType.DMA,
        pltpu.SemaphoreType.DMA,
        pltpu.SemaphoreType.DMA,
    ],
)
def _gather_kernel(xT, slab, out, idx_v, dst, shared, sem0, sem1, semm):
    sems = (sem0, sem1)

    sid = lax.axis_index("s")
    wid = sid * _NC + lax.axis_index("c")
    base = wid * _BPW
    sp = shared.at[sid]

    pltpu.sync_copy(xT.at[:, pl.ds(base, _BPW)], idx_v)

    tasks = [(p, c) for p in range(_NF // 2) for c in range(_BPW // _C)]

    def start(t, b):
        p, c = tasks[t]
        cp0 = pltpu.async_copy(
            slab.at[2 * p].at[idx_v.at[2 * p, pl.ds(c * _C, _C)]],
            dst.at[b, 0], sems[b])
        cp1 = pltpu.async_copy(
            slab.at[2 * p + 1].at[idx_v.at[2 * p + 1, pl.ds(c * _C, _C)]],
            dst.at[b, 1], sems[b])
        return (cp0, cp1)

    copies = [None, None]
    copies[0] = start(0, 0)
    for t in range(len(tasks)):
        b = t % 2
        if t + 1 < len(tasks):
            copies[(t + 1) % 2] = start(t + 1, (t + 1) % 2)
        copies[b][0].wait()
        copies[b][1].wait()
        m0 = pltpu.async_copy(dst.at[b, 0, :, pl.ds(0, _D)],
                              sp.at[:, pl.ds(0, _D)], semm)
        m1 = pltpu.async_copy(dst.at[b, 1, :, pl.ds(0, _D)],
                              sp.at[:, pl.ds(_D, _D)], semm)
        m0.wait()
        m1.wait()
        p, c = tasks[t]
        pltpu.sync_copy(
            sp, out.at[pl.ds(base + c * _C, _C), pl.ds(p * 2 * _D, 2 * _D)])


def kernel(x, table_0, table_1, table_2, table_3, table_4, table_5,
           table_6, table_7, table_8, table_9, table_10, table_11,
           table_12, table_13, table_14, table_15, table_16, table_17,
           table_18, table_19, table_20, table_21, table_22, table_23,
           table_24, table_25):
    tables = (table_0, table_1, table_2, table_3, table_4, table_5,
              table_6, table_7, table_8, table_9, table_10, table_11,
              table_12, table_13, table_14, table_15, table_16, table_17,
              table_18, table_19, table_20, table_21, table_22, table_23,
              table_24, table_25)
    slab = _compact_tc(*(t.T for t in tables))
    return _gather_kernel(x.T, slab)


# speedup vs baseline: 2.6344x; 1.0342x over previous
import functools

import jax
import jax.numpy as jnp
from jax import lax
from jax.experimental import pallas as pl
from jax.experimental.pallas import tpu as pltpu
from jax.experimental.pallas import tpu_sc as plsc

_NF = 26
_V = 100000
_D = 64
_B = 16384
_NC, _NS = 2, 16
_NW = _NC * _NS
_BPW = _B // _NW
_C = 128

_BS = 1024
_NBLK = (_V + _BS - 1) // _BS

_mesh = plsc.VectorSubcoreMesh(core_axis_name="c", subcore_axis_name="s")


def _compact_body(*refs):
    ins = refs[:_NF]
    x_in = refs[_NF]
    out = refs[_NF + 1]
    xt_out = refs[_NF + 2]
    for f in range(_NF):
        out[f, :, 0:_D] = ins[f][...].T

    @pl.when(pl.program_id(0) == 0)
    def _():
        xt_out[...] = x_in[...].T


_compact_tc = pl.pallas_call(
    _compact_body,
    grid=(_NBLK,),
    in_specs=[pl.BlockSpec((_D, _BS), lambda j: (0, j))] * _NF
    + [pl.BlockSpec((_B, _NF), lambda j: (0, 0))],
    out_specs=[
        pl.BlockSpec((_NF, _BS, 2 * _D), lambda j: (0, j, 0)),
        pl.BlockSpec((_NF, _B), lambda j: (0, 0)),
    ],
    out_shape=[
        jax.ShapeDtypeStruct((_NF, _V, 2 * _D), jnp.float32),
        jax.ShapeDtypeStruct((_NF, _B), jnp.int32),
    ],
)


@functools.partial(
    pl.kernel,
    out_type=jax.ShapeDtypeStruct((_B, _NF * _D), jnp.float32),
    mesh=_mesh,
    scratch_types=[
        pltpu.VMEM((_NF, _BPW), jnp.int32),
        pltpu.VMEM((2, 2, _C, 2 * _D), jnp.float32),
        pltpu.VMEM_SHARED((_NS, _C, 2 * _D), jnp.float32),
        pltpu.SemaphoreType.DMA,
        pltpu.SemaphoreType.DMA,
        pltpu.SemaphoreType.DMA,
    ],
)
def _gather_kernel(xT, slab, out, idx_v, dst, shared, sem0, sem1, semm):
    sems = (sem0, sem1)

    sid = lax.axis_index("s")
    wid = sid * _NC + lax.axis_index("c")
    base = wid * _BPW
    sp = shared.at[sid]

    pltpu.sync_copy(xT.at[:, pl.ds(base, _BPW)], idx_v)

    tasks = [(p, c) for p in range(_NF // 2) for c in range(_BPW // _C)]

    def start(t, b):
        p, c = tasks[t]
        cp0 = pltpu.async_copy(
            slab.at[2 * p].at[idx_v.at[2 * p, pl.ds(c * _C, _C)]],
            dst.at[b, 0], sems[b])
        cp1 = pltpu.async_copy(
            slab.at[2 * p + 1].at[idx_v.at[2 * p + 1, pl.ds(c * _C, _C)]],
            dst.at[b, 1], sems[b])
        return (cp0, cp1)

    copies = [None, None]
    copies[0] = start(0, 0)
    for t in range(len(tasks)):
        b = t % 2
        if t + 1 < len(tasks):
            copies[(t + 1) % 2] = start(t + 1, (t + 1) % 2)
        copies[b][0].wait()
        copies[b][1].wait()
        m0 = pltpu.async_copy(dst.at[b, 0, :, pl.ds(0, _D)],
                              sp.at[:, pl.ds(0, _D)], semm)
        m1 = pltpu.async_copy(dst.at[b, 1, :, pl.ds(0, _D)],
                              sp.at[:, pl.ds(_D, _D)], semm)
        m0.wait()
        m1.wait()
        p, c = tasks[t]
        pltpu.sync_copy(
            sp, out.at[pl.ds(base + c * _C, _C), pl.ds(p * 2 * _D, 2 * _D)])


def kernel(x, table_0, table_1, table_2, table_3, table_4, table_5,
           table_6, table_7, table_8, table_9, table_10, table_11,
           table_12, table_13, table_14, table_15, table_16, table_17,
           table_18, table_19, table_20, table_21, table_22, table_23,
           table_24, table_25):
    tables = (table_0, table_1, table_2, table_3, table_4, table_5,
              table_6, table_7, table_8, table_9, table_10, table_11,
              table_12, table_13, table_14, table_15, table_16, table_17,
              table_18, table_19, table_20, table_21, table_22, table_23,
              table_24, table_25)
    slab, xT = _compact_tc(*(t.T for t in tables), x)
    return _gather_kernel(xT, slab)


# speedup vs baseline: 2.8483x; 1.0812x over previous
import functools

import jax
import jax.numpy as jnp
from jax import lax
from jax.experimental import pallas as pl
from jax.experimental.pallas import tpu as pltpu
from jax.experimental.pallas import tpu_sc as plsc

_NF = 26
_V = 100000
_D = 64
_B = 16384
_NC, _NS = 2, 16
_NW = _NC * _NS
_BPW = _B // _NW
_C = 128

_BS = 1024
_NBLK = (_V + _BS - 1) // _BS

_mesh = plsc.VectorSubcoreMesh(core_axis_name="c", subcore_axis_name="s")


def _compact_body(*refs):
    ins = refs[:_NF]
    x_in = refs[_NF]
    out = refs[_NF + 1]
    xt_out = refs[_NF + 2]
    for p in range(_NF // 2):
        out[p, :, 0:_D] = ins[2 * p][...].T
        out[p, :, _D:2 * _D] = ins[2 * p + 1][...].T

    @pl.when(pl.program_id(0) == 0)
    def _():
        xt_out[...] = x_in[...].T


_compact_tc = pl.pallas_call(
    _compact_body,
    grid=(_NBLK,),
    in_specs=[pl.BlockSpec((_D, _BS), lambda j: (0, j))] * _NF
    + [pl.BlockSpec((_B, _NF), lambda j: (0, 0))],
    out_specs=[
        pl.BlockSpec((_NF // 2, _BS, 2 * _D), lambda j: (0, j, 0)),
        pl.BlockSpec((_NF, _B), lambda j: (0, 0)),
    ],
    out_shape=[
        jax.ShapeDtypeStruct((_NF // 2, _V, 2 * _D), jnp.float32),
        jax.ShapeDtypeStruct((_NF, _B), jnp.int32),
    ],
)


@functools.partial(
    pl.kernel,
    out_type=jax.ShapeDtypeStruct((_B, _NF * _D), jnp.float32),
    mesh=_mesh,
    scratch_types=[
        pltpu.VMEM((_NF, _BPW), jnp.int32),
        pltpu.VMEM((2, 2, _C, 2 * _D), jnp.float32),
        pltpu.VMEM_SHARED((_NS, _C, 2 * _D), jnp.float32),
        pltpu.SemaphoreType.DMA,
        pltpu.SemaphoreType.DMA,
        pltpu.SemaphoreType.DMA,
    ],
)
def _gather_kernel(xT, slab, out, idx_v, dst, shared, sem0, sem1, semm):
    sems = (sem0, sem1)

    sid = lax.axis_index("s")
    wid = sid * _NC + lax.axis_index("c")
    base = wid * _BPW
    sp = shared.at[sid]

    pltpu.sync_copy(xT.at[:, pl.ds(base, _BPW)], idx_v)

    tasks = [(p, c) for p in range(_NF // 2) for c in range(_BPW // _C)]

    def start(t, b):
        p, c = tasks[t]
        cp0 = pltpu.async_copy(
            slab.at[p].at[idx_v.at[2 * p, pl.ds(c * _C, _C)]],
            dst.at[b, 0], sems[b])
        cp1 = pltpu.async_copy(
            slab.at[p].at[idx_v.at[2 * p + 1, pl.ds(c * _C, _C)]],
            dst.at[b, 1], sems[b])
        return (cp0, cp1)

    copies = [None, None]
    copies[0] = start(0, 0)
    for t in range(len(tasks)):
        b = t % 2
        if t + 1 < len(tasks):
            copies[(t + 1) % 2] = start(t + 1, (t + 1) % 2)
        copies[b][0].wait()
        copies[b][1].wait()
        m0 = pltpu.async_copy(dst.at[b, 0, :, pl.ds(0, _D)],
                              sp.at[:, pl.ds(0, _D)], semm)
        m1 = pltpu.async_copy(dst.at[b, 1, :, pl.ds(_D, _D)],
                              sp.at[:, pl.ds(_D, _D)], semm)
        m0.wait()
        m1.wait()
        p, c = tasks[t]
        pltpu.sync_copy(
            sp, out.at[pl.ds(base + c * _C, _C), pl.ds(p * 2 * _D, 2 * _D)])


def kernel(x, table_0, table_1, table_2, table_3, table_4, table_5,
           table_6, table_7, table_8, table_9, table_10, table_11,
           table_12, table_13, table_14, table_15, table_16, table_17,
           table_18, table_19, table_20, table_21, table_22, table_23,
           table_24, table_25):
    tables = (table_0, table_1, table_2, table_3, table_4, table_5,
              table_6, table_7, table_8, table_9, table_10, table_11,
              table_12, table_13, table_14, table_15, table_16, table_17,
              table_18, table_19, table_20, table_21, table_22, table_23,
              table_24, table_25)
    slab, xT = _compact_tc(*(t.T for t in tables), x)
    return _gather_kernel(xT, slab)


# speedup vs baseline: 2.8731x; 1.0087x over previous
import functools

import jax
import jax.numpy as jnp
from jax import lax
from jax.experimental import pallas as pl
from jax.experimental.pallas import tpu as pltpu
from jax.experimental.pallas import tpu_sc as plsc

_NF = 26
_V = 100000
_D = 64
_B = 16384
_NC, _NS = 2, 16
_NW = _NC * _NS
_BPW = _B // _NW
_C = 128

_BS = 1536
_NBLK = (_V + _BS - 1) // _BS

_mesh = plsc.VectorSubcoreMesh(core_axis_name="c", subcore_axis_name="s")


def _compact_body(*refs):
    ins = refs[:_NF]
    x_in = refs[_NF]
    out = refs[_NF + 1]
    xt_out = refs[_NF + 2]
    for p in range(_NF // 2):
        out[p, :, 0:_D] = ins[2 * p][...].T
        out[p, :, _D:2 * _D] = ins[2 * p + 1][...].T

    @pl.when(pl.program_id(0) == 0)
    def _():
        xt_out[...] = x_in[...].T


_compact_tc = pl.pallas_call(
    _compact_body,
    grid=(_NBLK,),
    in_specs=[pl.BlockSpec((_D, _BS), lambda j: (0, j))] * _NF
    + [pl.BlockSpec((_B, _NF), lambda j: (0, 0))],
    out_specs=[
        pl.BlockSpec((_NF // 2, _BS, 2 * _D), lambda j: (0, j, 0)),
        pl.BlockSpec((_NF, _B), lambda j: (0, 0)),
    ],
    out_shape=[
        jax.ShapeDtypeStruct((_NF // 2, _V, 2 * _D), jnp.float32),
        jax.ShapeDtypeStruct((_NF, _B), jnp.int32),
    ],
)


@functools.partial(
    pl.kernel,
    out_type=jax.ShapeDtypeStruct((_B, _NF * _D), jnp.float32),
    mesh=_mesh,
    scratch_types=[
        pltpu.VMEM((_NF, _BPW), jnp.int32),
        pltpu.VMEM((2, 2, _C, 2 * _D), jnp.float32),
        pltpu.VMEM_SHARED((_NS, _C, 2 * _D), jnp.float32),
        pltpu.SemaphoreType.DMA,
        pltpu.SemaphoreType.DMA,
        pltpu.SemaphoreType.DMA,
    ],
)
def _gather_kernel(xT, slab, out, idx_v, dst, shared, sem0, sem1, semm):
    sems = (sem0, sem1)

    sid = lax.axis_index("s")
    wid = sid * _NC + lax.axis_index("c")
    base = wid * _BPW
    sp = shared.at[sid]

    pltpu.sync_copy(xT.at[:, pl.ds(base, _BPW)], idx_v)

    tasks = [(p, c) for p in range(_NF // 2) for c in range(_BPW // _C)]

    def start(t, b):
        p, c = tasks[t]
        cp0 = pltpu.async_copy(
            slab.at[p].at[idx_v.at[2 * p, pl.ds(c * _C, _C)]],
            dst.at[b, 0], sems[b])
        cp1 = pltpu.async_copy(
            slab.at[p].at[idx_v.at[2 * p + 1, pl.ds(c * _C, _C)]],
            dst.at[b, 1], sems[b])
        return (cp0, cp1)

    copies = [None, None]
    copies[0] = start(0, 0)
    for t in range(len(tasks)):
        b = t % 2
        if t + 1 < len(tasks):
            copies[(t + 1) % 2] = start(t + 1, (t + 1) % 2)
        copies[b][0].wait()
        copies[b][1].wait()
        m0 = pltpu.async_copy(dst.at[b, 0, :, pl.ds(0, _D)],
                              sp.at[:, pl.ds(0, _D)], semm)
        m1 = pltpu.async_copy(dst.at[b, 1, :, pl.ds(_D, _D)],
                              sp.at[:, pl.ds(_D, _D)], semm)
        m0.wait()
        m1.wait()
        p, c = tasks[t]
        pltpu.sync_copy(
            sp, out.at[pl.ds(base + c * _C, _C), pl.ds(p * 2 * _D, 2 * _D)])


def kernel(x, table_0, table_1, table_2, table_3, table_4, table_5,
           table_6, table_7, table_8, table_9, table_10, table_11,
           table_12, table_13, table_14, table_15, table_16, table_17,
           table_18, table_19, table_20, table_21, table_22, table_23,
           table_24, table_25):
    tables = (table_0, table_1, table_2, table_3, table_4, table_5,
              table_6, table_7, table_8, table_9, table_10, table_11,
              table_12, table_13, table_14, table_15, table_16, table_17,
              table_18, table_19, table_20, table_21, table_22, table_23,
              table_24, table_25)
    slab, xT = _compact_tc(*(t.T for t in tables), x)
    return _gather_kernel(xT, slab)
